# baseline (speedup 1.0000x reference)
import functools

import jax
import jax.numpy as jnp
from jax import lax
from jax.experimental import pallas as pl
from jax.experimental.pallas import tpu as pltpu
from jax.experimental.pallas import tpu_sc as plsc

L = 16
IDX_W = 128


def _make_composer(B, N, D):
    info = plsc.get_sparse_core_info()
    nw = info.num_cores * info.num_subcores
    chunk = B // nw
    n_gather = chunk // IDX_W

    mesh = plsc.VectorSubcoreMesh(core_axis_name="c", subcore_axis_name="s")

    @functools.partial(
        pl.kernel,
        mesh=mesh,
        compiler_params=pltpu.CompilerParams(use_tc_tiling_on_sc=False,
                                             needs_layout_passes=False),
        out_type=jax.ShapeDtypeStruct((B, D + D + 2), jnp.float32),
        scratch_types=[
            pltpu.VMEM((chunk, 3), jnp.int32),
            pltpu.VMEM((n_gather, IDX_W), jnp.int32),
            pltpu.VMEM((n_gather, IDX_W), jnp.int32),
            pltpu.VMEM((chunk, D), jnp.float32),
            pltpu.VMEM((chunk, D), jnp.float32),
            pltpu.VMEM((chunk, 2), jnp.float32),
            pltpu.SemaphoreType.DMA,
        ],
    )
    def composer(batch_hbm, s_hbm, out_hbm, batch_v, idx_obj, idx_nb,
                 rows_obj, rows_nb, tail_v, sem):
        wid = lax.axis_index("s") * info.num_cores + lax.axis_index("c")
        base = wid * chunk

        pltpu.sync_copy(batch_hbm.at[pl.ds(base, chunk)], batch_v)

        iota = lax.iota(jnp.int32, L)
        c0 = jnp.zeros((L,), jnp.int32)
        c1 = jnp.full((L,), 1, jnp.int32)
        c2 = jnp.full((L,), 2, jnp.int32)

        for j in range(chunk // L):
            r = iota + j * L
            o = plsc.load_gather(batch_v, [r, c0])
            n = plsc.load_gather(batch_v, [r, c1])
            t = plsc.load_gather(batch_v, [r, c2])
            idx_obj[j // (IDX_W // L), pl.ds((j % (IDX_W // L)) * L, L)] = o
            idx_nb[j // (IDX_W // L), pl.ds((j % (IDX_W // L)) * L, L)] = n
            plsc.store_scatter(tail_v, [r, c0], t.astype(jnp.float32))
            plsc.store_scatter(tail_v, [r, c1], o.astype(jnp.float32))

        handles = []
        for g in range(n_gather):
            handles.append(pltpu.async_copy(
                s_hbm.at[idx_obj.at[g]],
                rows_obj.at[pl.ds(g * IDX_W, IDX_W)], sem))
            handles.append(pltpu.async_copy(
                s_hbm.at[idx_nb.at[g]],
                rows_nb.at[pl.ds(g * IDX_W, IDX_W)], sem))
        for h in handles:
            h.wait()

        pltpu.sync_copy(rows_obj, out_hbm.at[pl.ds(base, chunk), pl.ds(0, D)])
        pltpu.sync_copy(rows_nb, out_hbm.at[pl.ds(base, chunk), pl.ds(D, D)])
        pltpu.sync_copy(tail_v, out_hbm.at[pl.ds(base, chunk), pl.ds(2 * D, 2)])

    return composer


def kernel(batch, S):
    B = batch.shape[0]
    N, D = S.shape
    return _make_composer(B, N, D)(batch, S)
